# Initial kernel scaffold; baseline (speedup 1.0000x reference)
import jax
import jax.numpy as jnp
from jax.experimental import pallas as pl


def kernel(hidden_states, router_logits, w13, w2):
    raise NotImplementedError("write your pallas kernel here")


# speedup vs baseline: 1.4393x; 1.4393x over previous
import jax
import jax.numpy as jnp
from jax.experimental import pallas as pl
from jax.experimental.pallas import tpu as pltpu

T, H, E, I = 128, 768, 64, 768


def _moe_body(logits_ref, hidden_ref, w13_ref, w2_ref, out_ref,
              i1_ref, i2_ref, w1_ref, w2w_ref):
    e = pl.program_id(0)

    @pl.when(e == 0)
    def _route():
        logits = logits_ref[...]
        m = jnp.max(logits, axis=1, keepdims=True)
        p = jnp.exp(logits - m)
        probs = p / jnp.sum(p, axis=1, keepdims=True)
        iota = jax.lax.broadcasted_iota(jnp.int32, (T, E), 1)
        m1 = jnp.max(probs, axis=1, keepdims=True)
        i1 = jnp.min(jnp.where(probs == m1, iota, E), axis=1, keepdims=True)
        oh1 = iota == i1
        pm = jnp.where(oh1, -jnp.inf, probs)
        m2 = jnp.max(pm, axis=1, keepdims=True)
        i2 = jnp.min(jnp.where(pm == m2, iota, E), axis=1, keepdims=True)
        denom = m1 + m2
        i1_ref[...] = i1
        i2_ref[...] = i2
        w1_ref[...] = m1 / denom
        w2w_ref[...] = m2 / denom
        out_ref[...] = jnp.zeros_like(out_ref)

    w13 = w13_ref[0]
    gate_up = jax.lax.dot_general(
        hidden_ref[...], w13, (((1,), (1,)), ((), ())),
        preferred_element_type=jnp.float32)
    gate = gate_up[:, :I]
    up = gate_up[:, I:]
    act = gate * jax.lax.logistic(gate) * up
    eo = jax.lax.dot_general(
        act, w2_ref[0], (((1,), (1,)), ((), ())),
        preferred_element_type=jnp.float32)
    col = (jnp.where(i1_ref[...] == e, w1_ref[...], 0.0)
           + jnp.where(i2_ref[...] == e, w2w_ref[...], 0.0))
    out_ref[...] += col * eo


def kernel(hidden_states, router_logits, w13, w2):
    return pl.pallas_call(
        _moe_body,
        grid=(E,),
        in_specs=[
            pl.BlockSpec((T, E), lambda e: (0, 0)),
            pl.BlockSpec((T, H), lambda e: (0, 0)),
            pl.BlockSpec((1, 2 * I, H), lambda e: (e, 0, 0)),
            pl.BlockSpec((1, H, I), lambda e: (e, 0, 0)),
        ],
        out_specs=pl.BlockSpec((T, H), lambda e: (0, 0)),
        out_shape=jax.ShapeDtypeStruct((T, H), jnp.float32),
        scratch_shapes=[
            pltpu.VMEM((T, 1), jnp.int32),
            pltpu.VMEM((T, 1), jnp.int32),
            pltpu.VMEM((T, 1), jnp.float32),
            pltpu.VMEM((T, 1), jnp.float32),
        ],
    )(router_logits, hidden_states, w13, w2)
